# Initial kernel scaffold; baseline (speedup 1.0000x reference)
import jax
import jax.numpy as jnp
from jax.experimental import pallas as pl


def kernel(cls_logits, labels, matched_idxs):
    raise NotImplementedError("write your pallas kernel here")


# speedup vs baseline: 7.2745x; 7.2745x over previous
import functools

import jax
import jax.numpy as jnp
from jax import lax
from jax.experimental import pallas as pl
from jax.experimental.pallas import tpu as pltpu
from jax.experimental.pallas import tpu_sc as plsc

BETWEEN_THRESHOLD = -2
ALPHA = 0.25
GAMMA = 2.0

_NC = 2
_NS = 16
_NW = _NC * _NS
_L = 16

_B, _A, _C, _G = 4, 120000, 80, 100
_TA = 960
_NB = _A // _TA
_A_PAD = 122880
_CH = _A_PAD // _NW
_NV = _CH // _L


def _sc_body(matched_hbm, labels_hbm, out_hbm, m_v, o_v, lab_v):
    wid = lax.axis_index("s") * _NC + lax.axis_index("c")
    pltpu.sync_copy(labels_hbm, lab_v)
    for b in range(_B):
        base = b * _A_PAD + wid * _CH
        pltpu.sync_copy(matched_hbm.at[pl.ds(base, _CH)], m_v)

        def body(i, carry):
            m = m_v[pl.ds(i * _L, _L)]
            fg = m >= 0
            safe_idx = jnp.where(fg, m + b * _G, 0)
            val = plsc.load_gather(lab_v, [safe_idx])
            gt = jnp.where(fg, val, jnp.where(m == BETWEEN_THRESHOLD, -2, -1))
            o_v[pl.ds(i * _L, _L)] = gt
            return carry

        lax.fori_loop(0, _NV, body, 0)
        pltpu.sync_copy(o_v, out_hbm.at[pl.ds(base, _CH)])


@functools.cache
def _sc_assign():
    return pl.kernel(
        _sc_body,
        out_type=jax.ShapeDtypeStruct((_B * _A_PAD,), jnp.int32),
        mesh=plsc.VectorSubcoreMesh(
            core_axis_name="c", subcore_axis_name="s",
            num_cores=_NC, num_subcores=_NS,
        ),
        scratch_types=[
            pltpu.VMEM((_CH,), jnp.int32),
            pltpu.VMEM((_CH,), jnp.int32),
            pltpu.VMEM((_B * _G,), jnp.int32),
        ],
        compiler_params=pltpu.CompilerParams(needs_layout_passes=False),
    )


def _tc_body(x_ref, gt_ref, sum_ref, cnt_ref):
    i = pl.program_id(1)
    x = x_ref[0]
    g = gt_ref[0, 0, 0].reshape(_TA, 1)

    col = lax.broadcasted_iota(jnp.int32, (_TA, _C), 1)
    mask = col == g
    valid = (g != BETWEEN_THRESHOLD).astype(jnp.float32)
    fg = g >= 0

    e = jnp.exp(-jnp.abs(x))
    sp = jnp.maximum(x, 0.0) + jnp.log1p(e)
    r = 1.0 / (1.0 + e)
    s = jnp.where(x >= 0, r, e * r)
    loss0 = (1.0 - ALPHA) * sp * (s * s)
    one_m_s = 1.0 - s
    loss1 = ALPHA * (sp - x) * (one_m_s * one_m_s)
    elem = jnp.where(mask, loss1, loss0)

    bsum = jnp.sum(elem * valid).reshape(1, 1)
    bcnt = jnp.sum(jnp.where(fg, 1.0, 0.0)).reshape(1, 1)

    @pl.when(i == 0)
    def _init():
        sum_ref[0] = bsum
        cnt_ref[0] = bcnt

    @pl.when(i > 0)
    def _acc():
        sum_ref[0] = sum_ref[0] + bsum
        cnt_ref[0] = cnt_ref[0] + bcnt


_tc_loss = pl.pallas_call(
    _tc_body,
    grid=(_B, _NB),
    in_specs=[
        pl.BlockSpec((1, _TA, _C), lambda b, i: (b, i, 0)),
        pl.BlockSpec((1, 1, 1, _TA), lambda b, i: (b, i, 0, 0)),
    ],
    out_specs=[
        pl.BlockSpec((1, 1, 1), lambda b, i: (b, 0, 0)),
        pl.BlockSpec((1, 1, 1), lambda b, i: (b, 0, 0)),
    ],
    out_shape=[
        jax.ShapeDtypeStruct((_B, 1, 1), jnp.float32),
        jax.ShapeDtypeStruct((_B, 1, 1), jnp.float32),
    ],
)


def kernel(cls_logits, labels, matched_idxs):
    B, A, C = cls_logits.shape
    pad = jnp.full((B, _A_PAD - A), BETWEEN_THRESHOLD, dtype=jnp.int32)
    matched_pad = jnp.concatenate([matched_idxs, pad], axis=1).reshape(-1)
    labels_flat = labels.reshape(-1)

    gt_flat = _sc_assign()(matched_pad, labels_flat)
    gt4 = gt_flat.reshape(B, _A_PAD // _TA, 1, _TA)

    sums, cnts = _tc_loss(cls_logits, gt4)
    sums = sums.reshape(B)
    cnts = cnts.reshape(B)
    losses = sums / jnp.maximum(1.0, cnts)
    return losses.sum() / B


# speedup vs baseline: 10.4611x; 1.4380x over previous
import functools

import jax
import jax.numpy as jnp
from jax import lax
from jax.experimental import pallas as pl
from jax.experimental.pallas import tpu as pltpu
from jax.experimental.pallas import tpu_sc as plsc

BETWEEN_THRESHOLD = -2
ALPHA = 0.25
GAMMA = 2.0

_NC = 2
_NS = 16
_NW = _NC * _NS
_L = 16

_B, _A, _C, _G = 4, 120000, 80, 100
_TA = 4800
_NB = _A // _TA
_A_PAD = 122880
_CH = _A_PAD // _NW
_NV = _CH // _L


def _sc_body(matched_hbm, labels_hbm, out_hbm, m_v, o_v, lab_v):
    wid = lax.axis_index("s") * _NC + lax.axis_index("c")
    pltpu.sync_copy(labels_hbm, lab_v)
    for b in range(_B):
        base = b * _A_PAD + wid * _CH
        pltpu.sync_copy(matched_hbm.at[pl.ds(base, _CH)], m_v)

        def body(i, carry):
            m = m_v[pl.ds(i * _L, _L)]
            fg = m >= 0
            safe_idx = jnp.where(fg, m + b * _G, 0)
            val = plsc.load_gather(lab_v, [safe_idx])
            gt = jnp.where(fg, val, jnp.where(m == BETWEEN_THRESHOLD, -2, -1))
            o_v[pl.ds(i * _L, _L)] = gt
            return carry

        lax.fori_loop(0, _NV, body, 0)
        pltpu.sync_copy(o_v, out_hbm.at[pl.ds(base, _CH)])


@functools.cache
def _sc_assign():
    return pl.kernel(
        _sc_body,
        out_type=jax.ShapeDtypeStruct((_B * _A_PAD,), jnp.int32),
        mesh=plsc.VectorSubcoreMesh(
            core_axis_name="c", subcore_axis_name="s",
            num_cores=_NC, num_subcores=_NS,
        ),
        scratch_types=[
            pltpu.VMEM((_CH,), jnp.int32),
            pltpu.VMEM((_CH,), jnp.int32),
            pltpu.VMEM((_B * _G,), jnp.int32),
        ],
        compiler_params=pltpu.CompilerParams(needs_layout_passes=False),
    )


def _tc_body(x_ref, gt_ref, sum_ref, cnt_ref):
    i = pl.program_id(1)
    x = x_ref[0]
    g = gt_ref[0, 0, 0].reshape(_TA, 1)

    col = lax.broadcasted_iota(jnp.int32, (_TA, _C), 1)
    mask = col == g
    valid = (g != BETWEEN_THRESHOLD).astype(jnp.float32)
    fg = g >= 0

    e = jnp.exp(-jnp.abs(x))
    q = 1.0 + e
    l = jnp.log(q)
    z = jnp.where(mask, -x, x)
    sp = jnp.maximum(z, 0.0) + l
    r = 1.0 / q
    s = jnp.where(z >= 0, r, e * r)
    w = jnp.where(mask, ALPHA, 1.0 - ALPHA) * valid
    elem = w * sp * (s * s)

    bsum = jnp.sum(elem).reshape(1, 1)
    bcnt = jnp.sum(jnp.where(fg, 1.0, 0.0)).reshape(1, 1)

    @pl.when(i == 0)
    def _init():
        sum_ref[0] = bsum
        cnt_ref[0] = bcnt

    @pl.when(i > 0)
    def _acc():
        sum_ref[0] = sum_ref[0] + bsum
        cnt_ref[0] = cnt_ref[0] + bcnt


_tc_loss = pl.pallas_call(
    _tc_body,
    grid=(_B, _NB),
    in_specs=[
        pl.BlockSpec((1, _TA, _C), lambda b, i: (b, i, 0)),
        pl.BlockSpec((1, 1, 1, _TA), lambda b, i: (b, i, 0, 0)),
    ],
    out_specs=[
        pl.BlockSpec((1, 1, 1), lambda b, i: (b, 0, 0)),
        pl.BlockSpec((1, 1, 1), lambda b, i: (b, 0, 0)),
    ],
    out_shape=[
        jax.ShapeDtypeStruct((_B, 1, 1), jnp.float32),
        jax.ShapeDtypeStruct((_B, 1, 1), jnp.float32),
    ],
)


def kernel(cls_logits, labels, matched_idxs):
    B, A, C = cls_logits.shape
    pad = jnp.full((B, _A_PAD - A), BETWEEN_THRESHOLD, dtype=jnp.int32)
    matched_pad = jnp.concatenate([matched_idxs, pad], axis=1).reshape(-1)
    labels_flat = labels.reshape(-1)

    gt_flat = _sc_assign()(matched_pad, labels_flat)
    gt4 = gt_flat.reshape(B, _A_PAD)[:, :A].reshape(B, _NB, 1, _TA)

    sums, cnts = _tc_loss(cls_logits, gt4)
    sums = sums.reshape(B)
    cnts = cnts.reshape(B)
    losses = sums / jnp.maximum(1.0, cnts)
    return losses.sum() / B
